# baseline (speedup 1.0000x reference)
import dataclasses
import functools

import jax
import jax.numpy as jnp
from jax import lax
from jax.experimental import pallas as pl
from jax.experimental.pallas import tpu as pltpu
from jax.experimental.pallas import tpu_sc as plsc

R = 128
C = 32768
L = 16
NC = 2
NS = 16
NW = NC * NS

R_SC = 64
R_TC = R - R_SC
RPW = R_SC // NW
ZPW = R_TC // NW

SC_BLK = 16
SC_NBLK = C // (SC_BLK * L)
PIECES = 4
PIECE = C // PIECES
BLK_PER_PIECE = SC_NBLK // PIECES

TC_BLK = 4096
TC_NBLK = C // TC_BLK
TC_TILES = TC_BLK // 128
BIG = 1 << 30


def _sc_piece_scan(buf, q, carry):
    @plsc.parallel_loop(
        q * BLK_PER_PIECE, (q + 1) * BLK_PER_PIECE, step=1, unroll=2, carry=carry
    )
    def carry_out(t, c):
        m, blk = c
        base = t * (SC_BLK * L)
        vs = [buf[pl.ds(base + j * L, L)] for j in range(SC_BLK)]
        while len(vs) > 1:
            vs = [jnp.maximum(vs[i], vs[i + 1]) for i in range(0, len(vs), 2)]
        bm = vs[0]
        changed = bm > m
        m = jnp.maximum(m, bm)
        blk = jnp.where(changed, t, blk)
        return m, blk

    return carry_out


def _sc_row_resolve(buf, lane, m, blk):
    base_idx = blk * (SC_BLK * L) + lane
    j_in_blk = jnp.zeros((L,), dtype=jnp.int32)
    for j in range(SC_BLK - 1, -1, -1):
        v = plsc.load_gather(buf, [base_idx + j * L])
        j_in_blk = jnp.where(v == m, j, j_in_blk)
    elem_idx = base_idx + j_in_blk * L

    best = jnp.max(m)
    cand = jnp.where(m == best, elem_idx, BIG)
    return jnp.min(cand)


def _sc_argmax(logits):
    mesh = plsc.VectorSubcoreMesh(
        core_axis_name="c", subcore_axis_name="s", num_cores=NC, num_subcores=NS
    )
    cp = pltpu.CompilerParams()
    if "needs_layout_passes" in pltpu.CompilerParams.__dataclass_fields__:
        cp = dataclasses.replace(cp, needs_layout_passes=False)

    @functools.partial(
        pl.kernel,
        out_type=jax.ShapeDtypeStruct((R, L), jnp.int32),
        mesh=mesh,
        compiler_params=cp,
        scratch_types=[
            pltpu.VMEM((C,), jnp.float32),
            pltpu.VMEM((C,), jnp.float32),
            pltpu.VMEM((L,), jnp.int32),
            pltpu.SemaphoreType.DMA,
            pltpu.SemaphoreType.DMA,
        ],
    )
    def k(x_hbm, out_hbm, buf_a, buf_b, res_v, sem_a, sem_b):
        wid = lax.axis_index("s") * NC + lax.axis_index("c")
        row0 = wid * RPW
        bufs = (buf_a, buf_b)
        sems = (sem_a, sem_b)
        lane = lax.iota(jnp.int32, L)

        def start_piece(g):
            r, q = divmod(g, PIECES)
            return pltpu.async_copy(
                x_hbm.at[pl.ds((row0 + r) * C + q * PIECE, PIECE)],
                bufs[r % 2].at[pl.ds(q * PIECE, PIECE)],
                sems[r % 2],
            )

        AHEAD = 2
        total = RPW * PIECES
        copies = [start_piece(g) for g in range(min(AHEAD, total))]
        res_v[...] = jnp.zeros((L,), dtype=jnp.int32)
        for z in range(ZPW):
            pltpu.sync_copy(res_v, out_hbm.at[R_SC + wid * ZPW + z])

        for r in range(RPW):
            carry = (
                jnp.full((L,), -jnp.inf, dtype=jnp.float32),
                jnp.zeros((L,), dtype=jnp.int32),
            )
            for q in range(PIECES):
                g = r * PIECES + q
                copies[g].wait()
                if g + AHEAD < total:
                    copies.append(start_piece(g + AHEAD))
                carry = _sc_piece_scan(bufs[r % 2], q, carry)
            best_idx = _sc_row_resolve(bufs[r % 2], lane, *carry)
            res_v[...] = jnp.where(lane == 0, best_idx, 0)
            pltpu.sync_copy(res_v, out_hbm.at[row0 + r])

    return k(logits.reshape(R * C))


def _tc_argmax(x):
    def body(x_ref, i_ref, m_a, i_a, m_b, i_b):
        k = pl.program_id(0)

        @pl.when(k == 0)
        def _():
            m_a[...] = jnp.full((R_TC, 128), -jnp.inf, dtype=jnp.float32)
            i_a[...] = jnp.zeros((R_TC, 128), dtype=jnp.int32)
            m_b[...] = jnp.full((R_TC, 128), -jnp.inf, dtype=jnp.float32)
            i_b[...] = jnp.zeros((R_TC, 128), dtype=jnp.int32)

        ma, ia = m_a[...], i_a[...]
        mb, ib = m_b[...], i_b[...]
        for t in range(TC_TILES):
            tile = x_ref[:, t * 128:(t + 1) * 128]
            c = k * TC_TILES + t
            if t % 2 == 0:
                changed = tile > ma
                ma = jnp.maximum(ma, tile)
                ia = jnp.where(changed, c, ia)
            else:
                changed = tile > mb
                mb = jnp.maximum(mb, tile)
                ib = jnp.where(changed, c, ib)
        m_a[...], i_a[...] = ma, ia
        m_b[...], i_b[...] = mb, ib

        @pl.when(k == TC_NBLK - 1)
        def _():
            take_b = (mb > ma) | ((mb == ma) & (ib < ia))
            m = jnp.where(take_b, mb, ma)
            idx = jnp.where(take_b, ib, ia)
            lane = lax.broadcasted_iota(jnp.int32, (R_TC, 128), 1)
            gidx = idx * 128 + lane
            best = jnp.max(m, axis=1, keepdims=True)
            cand = jnp.where(m == best, gidx, BIG)
            res = jnp.min(cand, axis=1, keepdims=True)
            i_ref[...] = jnp.concatenate(
                [jnp.zeros((R_SC, 1), dtype=jnp.int32), res], axis=0
            )

    i = pl.pallas_call(
        body,
        grid=(TC_NBLK,),
        in_specs=[pl.BlockSpec((R_TC, TC_BLK), lambda k: (1, k))],
        out_specs=pl.BlockSpec((R, 1), lambda k: (0, 0)),
        out_shape=jax.ShapeDtypeStruct((R, 1), jnp.int32),
        scratch_shapes=[
            pltpu.VMEM((R_TC, 128), jnp.float32),
            pltpu.VMEM((R_TC, 128), jnp.int32),
            pltpu.VMEM((R_TC, 128), jnp.float32),
            pltpu.VMEM((R_TC, 128), jnp.int32),
        ],
    )(x)
    return i


def kernel(logits):
    staging = _sc_argmax(logits)
    tc_idx = _tc_argmax(logits)
    return staging[:, 0] + tc_idx[:, 0]


# speedup vs baseline: 1.6038x; 1.6038x over previous
import dataclasses
import functools

import jax
import jax.numpy as jnp
from jax import lax
from jax.experimental import pallas as pl
from jax.experimental.pallas import tpu as pltpu
from jax.experimental.pallas import tpu_sc as plsc

R = 128
C = 32768
L = 16
NC = 2
NS = 16
NW = NC * NS

R_SC = 64
R_TC = R - R_SC
RPW = R_SC // NW
ZPW = R_TC // NW

SC_BLK = 16
SC_NBLK = C // (SC_BLK * L)
PIECES = 4
PIECE = C // PIECES
BLK_PER_PIECE = SC_NBLK // PIECES

TC_BLK = 4096
TC_NBLK = C // TC_BLK
TC_TILES = TC_BLK // 128
BIG = 1 << 30


def _sc_piece_scan(buf, q, carry):
    @plsc.parallel_loop(
        q * BLK_PER_PIECE, (q + 1) * BLK_PER_PIECE, step=1, unroll=2, carry=carry
    )
    def carry_out(t, c):
        m, blk = c
        base = t * (SC_BLK * L)
        vs = [buf[pl.ds(base + j * L, L)] for j in range(SC_BLK)]
        while len(vs) > 1:
            vs = [jnp.maximum(vs[i], vs[i + 1]) for i in range(0, len(vs), 2)]
        bm = vs[0]
        changed = bm > m
        m = jnp.maximum(m, bm)
        blk = jnp.where(changed, t, blk)
        return m, blk

    return carry_out


def _sc_row_resolve(buf, lane, m, blk):
    base_idx = blk * (SC_BLK * L) + lane
    j_in_blk = jnp.zeros((L,), dtype=jnp.int32)
    for j in range(SC_BLK - 1, -1, -1):
        v = plsc.load_gather(buf, [base_idx + j * L])
        j_in_blk = jnp.where(v == m, j, j_in_blk)
    elem_idx = base_idx + j_in_blk * L

    best = jnp.max(m)
    cand = jnp.where(m == best, elem_idx, BIG)
    return jnp.min(cand)


def _sc_argmax(logits):
    mesh = plsc.VectorSubcoreMesh(
        core_axis_name="c", subcore_axis_name="s", num_cores=NC, num_subcores=NS
    )
    cp = pltpu.CompilerParams()
    if "needs_layout_passes" in pltpu.CompilerParams.__dataclass_fields__:
        cp = dataclasses.replace(cp, needs_layout_passes=False)

    @functools.partial(
        pl.kernel,
        out_type=jax.ShapeDtypeStruct((R, L), jnp.int32),
        mesh=mesh,
        compiler_params=cp,
        scratch_types=[
            pltpu.VMEM((C,), jnp.float32),
            pltpu.VMEM((C,), jnp.float32),
            pltpu.VMEM((L,), jnp.int32),
            pltpu.SemaphoreType.DMA,
            pltpu.SemaphoreType.DMA,
        ],
    )
    def k(x_hbm, out_hbm, buf_a, buf_b, res_v, sem_a, sem_b):
        wid = lax.axis_index("s") * NC + lax.axis_index("c")
        row0 = wid * RPW
        bufs = (buf_a, buf_b)
        sems = (sem_a, sem_b)
        lane = lax.iota(jnp.int32, L)

        def start_piece(g):
            r, q = divmod(g, PIECES)
            return pltpu.async_copy(
                x_hbm.at[row0 + r, pl.ds(q * PIECE, PIECE)],
                bufs[r % 2].at[pl.ds(q * PIECE, PIECE)],
                sems[r % 2],
            )

        AHEAD = 2
        total = RPW * PIECES
        copies = [start_piece(g) for g in range(min(AHEAD, total))]
        res_v[...] = jnp.zeros((L,), dtype=jnp.int32)
        for z in range(ZPW):
            pltpu.sync_copy(res_v, out_hbm.at[R_SC + wid * ZPW + z])

        for r in range(RPW):
            carry = (
                jnp.full((L,), -jnp.inf, dtype=jnp.float32),
                jnp.zeros((L,), dtype=jnp.int32),
            )
            for q in range(PIECES):
                g = r * PIECES + q
                copies[g].wait()
                if g + AHEAD < total:
                    copies.append(start_piece(g + AHEAD))
                carry = _sc_piece_scan(bufs[r % 2], q, carry)
            best_idx = _sc_row_resolve(bufs[r % 2], lane, *carry)
            res_v[...] = jnp.where(lane == 0, best_idx, 0)
            pltpu.sync_copy(res_v, out_hbm.at[row0 + r])

    return k(logits)


def _tc_argmax(x):
    def body(x_ref, i_ref, m_a, i_a, m_b, i_b):
        k = pl.program_id(0)

        @pl.when(k == 0)
        def _():
            m_a[...] = jnp.full((R_TC, 128), -jnp.inf, dtype=jnp.float32)
            i_a[...] = jnp.zeros((R_TC, 128), dtype=jnp.int32)
            m_b[...] = jnp.full((R_TC, 128), -jnp.inf, dtype=jnp.float32)
            i_b[...] = jnp.zeros((R_TC, 128), dtype=jnp.int32)

        ma, ia = m_a[...], i_a[...]
        mb, ib = m_b[...], i_b[...]
        for t in range(TC_TILES):
            tile = x_ref[:, t * 128:(t + 1) * 128]
            c = k * TC_TILES + t
            if t % 2 == 0:
                changed = tile > ma
                ma = jnp.maximum(ma, tile)
                ia = jnp.where(changed, c, ia)
            else:
                changed = tile > mb
                mb = jnp.maximum(mb, tile)
                ib = jnp.where(changed, c, ib)
        m_a[...], i_a[...] = ma, ia
        m_b[...], i_b[...] = mb, ib

        @pl.when(k == TC_NBLK - 1)
        def _():
            take_b = (mb > ma) | ((mb == ma) & (ib < ia))
            m = jnp.where(take_b, mb, ma)
            idx = jnp.where(take_b, ib, ia)
            lane = lax.broadcasted_iota(jnp.int32, (R_TC, 128), 1)
            gidx = idx * 128 + lane
            best = jnp.max(m, axis=1, keepdims=True)
            cand = jnp.where(m == best, gidx, BIG)
            res = jnp.min(cand, axis=1, keepdims=True)
            i_ref[...] = jnp.concatenate(
                [jnp.zeros((R_SC, 1), dtype=jnp.int32), res], axis=0
            )

    i = pl.pallas_call(
        body,
        grid=(TC_NBLK,),
        in_specs=[pl.BlockSpec((R_TC, TC_BLK), lambda k: (1, k))],
        out_specs=pl.BlockSpec((R, 1), lambda k: (0, 0)),
        out_shape=jax.ShapeDtypeStruct((R, 1), jnp.int32),
        scratch_shapes=[
            pltpu.VMEM((R_TC, 128), jnp.float32),
            pltpu.VMEM((R_TC, 128), jnp.int32),
            pltpu.VMEM((R_TC, 128), jnp.float32),
            pltpu.VMEM((R_TC, 128), jnp.int32),
        ],
    )(x)
    return i


def kernel(logits):
    staging = _sc_argmax(logits)
    tc_idx = _tc_argmax(logits)
    return staging[:, 0] + tc_idx[:, 0]


# speedup vs baseline: 1.6188x; 1.0093x over previous
import dataclasses
import functools

import jax
import jax.numpy as jnp
from jax import lax
from jax.experimental import pallas as pl
from jax.experimental.pallas import tpu as pltpu
from jax.experimental.pallas import tpu_sc as plsc

R = 128
C = 32768
L = 16
NC = 2
NS = 16
NW = NC * NS

R_SC = 64
R_TC = R - R_SC
RPW = R_SC // NW
ZPW = R_TC // NW

SC_BLK = 16
SC_NBLK = C // (SC_BLK * L)
PIECES = 1
PIECE = C // PIECES
BLK_PER_PIECE = SC_NBLK // PIECES

TC_BLK = 8192
TC_NBLK = C // TC_BLK
TC_TILES = TC_BLK // 128
BIG = 1 << 30


def _sc_piece_scan(buf, q, carry):
    @plsc.parallel_loop(
        q * BLK_PER_PIECE, (q + 1) * BLK_PER_PIECE, step=1, unroll=2, carry=carry
    )
    def carry_out(t, c):
        m, blk = c
        base = t * (SC_BLK * L)
        vs = [buf[pl.ds(base + j * L, L)] for j in range(SC_BLK)]
        while len(vs) > 1:
            vs = [jnp.maximum(vs[i], vs[i + 1]) for i in range(0, len(vs), 2)]
        bm = vs[0]
        changed = bm > m
        m = jnp.maximum(m, bm)
        blk = jnp.where(changed, t, blk)
        return m, blk

    return carry_out


def _sc_row_resolve(buf, lane, m, blk):
    base_idx = blk * (SC_BLK * L) + lane
    j_in_blk = jnp.zeros((L,), dtype=jnp.int32)
    for j in range(SC_BLK - 1, -1, -1):
        v = plsc.load_gather(buf, [base_idx + j * L])
        j_in_blk = jnp.where(v == m, j, j_in_blk)
    elem_idx = base_idx + j_in_blk * L

    best = jnp.max(m)
    cand = jnp.where(m == best, elem_idx, BIG)
    return jnp.min(cand)


def _sc_argmax(logits):
    mesh = plsc.VectorSubcoreMesh(
        core_axis_name="c", subcore_axis_name="s", num_cores=NC, num_subcores=NS
    )
    cp = pltpu.CompilerParams()
    if "needs_layout_passes" in pltpu.CompilerParams.__dataclass_fields__:
        cp = dataclasses.replace(cp, needs_layout_passes=False)

    @functools.partial(
        pl.kernel,
        out_type=jax.ShapeDtypeStruct((R, L), jnp.int32),
        mesh=mesh,
        compiler_params=cp,
        scratch_types=[
            pltpu.VMEM((C,), jnp.float32),
            pltpu.VMEM((C,), jnp.float32),
            pltpu.VMEM((L,), jnp.int32),
            pltpu.SemaphoreType.DMA,
            pltpu.SemaphoreType.DMA,
        ],
    )
    def k(x_hbm, out_hbm, buf_a, buf_b, res_v, sem_a, sem_b):
        wid = lax.axis_index("s") * NC + lax.axis_index("c")
        row0 = wid * RPW
        bufs = (buf_a, buf_b)
        sems = (sem_a, sem_b)
        lane = lax.iota(jnp.int32, L)

        def start_piece(g):
            r, q = divmod(g, PIECES)
            return pltpu.async_copy(
                x_hbm.at[row0 + r, pl.ds(q * PIECE, PIECE)],
                bufs[r % 2].at[pl.ds(q * PIECE, PIECE)],
                sems[r % 2],
            )

        AHEAD = 2
        total = RPW * PIECES
        copies = [start_piece(g) for g in range(min(AHEAD, total))]
        res_v[...] = jnp.zeros((L,), dtype=jnp.int32)
        for z in range(ZPW):
            pltpu.sync_copy(res_v, out_hbm.at[R_SC + wid * ZPW + z])

        for r in range(RPW):
            carry = (
                jnp.full((L,), -jnp.inf, dtype=jnp.float32),
                jnp.zeros((L,), dtype=jnp.int32),
            )
            for q in range(PIECES):
                g = r * PIECES + q
                copies[g].wait()
                if g + AHEAD < total:
                    copies.append(start_piece(g + AHEAD))
                carry = _sc_piece_scan(bufs[r % 2], q, carry)
            best_idx = _sc_row_resolve(bufs[r % 2], lane, *carry)
            res_v[...] = jnp.where(lane == 0, best_idx, 0)
            pltpu.sync_copy(res_v, out_hbm.at[row0 + r])

    return k(logits)


def _tc_argmax(x):
    def body(x_ref, i_ref, m_a, i_a, m_b, i_b):
        k = pl.program_id(0)

        @pl.when(k == 0)
        def _():
            m_a[...] = jnp.full((R_TC, 128), -jnp.inf, dtype=jnp.float32)
            i_a[...] = jnp.zeros((R_TC, 128), dtype=jnp.int32)
            m_b[...] = jnp.full((R_TC, 128), -jnp.inf, dtype=jnp.float32)
            i_b[...] = jnp.zeros((R_TC, 128), dtype=jnp.int32)

        ma, ia = m_a[...], i_a[...]
        mb, ib = m_b[...], i_b[...]
        for t in range(TC_TILES):
            tile = x_ref[:, t * 128:(t + 1) * 128]
            c = k * TC_TILES + t
            if t % 2 == 0:
                changed = tile > ma
                ma = jnp.maximum(ma, tile)
                ia = jnp.where(changed, c, ia)
            else:
                changed = tile > mb
                mb = jnp.maximum(mb, tile)
                ib = jnp.where(changed, c, ib)
        m_a[...], i_a[...] = ma, ia
        m_b[...], i_b[...] = mb, ib

        @pl.when(k == TC_NBLK - 1)
        def _():
            take_b = (mb > ma) | ((mb == ma) & (ib < ia))
            m = jnp.where(take_b, mb, ma)
            idx = jnp.where(take_b, ib, ia)
            lane = lax.broadcasted_iota(jnp.int32, (R_TC, 128), 1)
            gidx = idx * 128 + lane
            best = jnp.max(m, axis=1, keepdims=True)
            cand = jnp.where(m == best, gidx, BIG)
            res = jnp.min(cand, axis=1, keepdims=True)
            i_ref[...] = jnp.concatenate(
                [jnp.zeros((R_SC, 1), dtype=jnp.int32), res], axis=0
            )

    i = pl.pallas_call(
        body,
        grid=(TC_NBLK,),
        in_specs=[pl.BlockSpec((R_TC, TC_BLK), lambda k: (1, k))],
        out_specs=pl.BlockSpec((R, 1), lambda k: (0, 0)),
        out_shape=jax.ShapeDtypeStruct((R, 1), jnp.int32),
        scratch_shapes=[
            pltpu.VMEM((R_TC, 128), jnp.float32),
            pltpu.VMEM((R_TC, 128), jnp.int32),
            pltpu.VMEM((R_TC, 128), jnp.float32),
            pltpu.VMEM((R_TC, 128), jnp.int32),
        ],
    )(x)
    return i


def kernel(logits):
    staging = _sc_argmax(logits)
    tc_idx = _tc_argmax(logits)
    return staging[:, 0] + tc_idx[:, 0]
